# baseline (speedup 1.0000x reference)
import jax
import jax.numpy as jnp
from jax.experimental import pallas as pl
from jax.experimental.pallas import tpu as pltpu

L, C, N, M, BITS = 100, 100, 13000, 200000, 64
BLK = 8000


def _body(tl_ref, trl_ref, tc_ref, db_ref, aB_ref, avg_ref, zero_ref, freq_ref):
    j = pl.program_id(0)

    @pl.when(j == 0)
    def _init():
        tl = tl_ref[...].astype(jnp.bfloat16)
        trl = trl_ref[...].astype(jnp.bfloat16)
        simd = jax.lax.dot_general(tl, trl, (((1,), (1,)), ((), ())),
                                   preferred_element_type=jnp.float32)
        sim = (simd > 0.0).astype(jnp.bfloat16)
        svote = jax.lax.dot_general(sim, tc_ref[...].astype(jnp.bfloat16),
                                    (((1,), (0,)), ((), ())),
                                    preferred_element_type=jnp.float32)
        aB_ref[...] = jnp.sign(svote)
        freq_ref[...] = jnp.zeros_like(freq_ref)

    aBb = aB_ref[...].astype(jnp.bfloat16)
    db = db_ref[...].astype(jnp.bfloat16)
    matc = jax.lax.dot_general(aBb, db, (((1,), (1,)), ((), ())),
                               preferred_element_type=jnp.float32)
    freq_ref[...] += jnp.sum((matc == float(BITS)).astype(jnp.float32),
                             axis=1, keepdims=True)

    @pl.when(j == pl.num_programs(0) - 1)
    def _fini():
        freq = freq_ref[...]
        avg_ref[...] = (jnp.sum(freq) / float(L)).reshape(1, 1)
        zero_ref[...] = jnp.sum((freq == 0.0).astype(jnp.float32)).reshape(1, 1)


def kernel(traincodes, dB, target_labels, trainlabels):
    nsteps = M // BLK
    aB, avg, zero = pl.pallas_call(
        _body,
        grid=(nsteps,),
        in_specs=[
            pl.BlockSpec((L, C), lambda j: (0, 0)),
            pl.BlockSpec((N, C), lambda j: (0, 0)),
            pl.BlockSpec((N, BITS), lambda j: (0, 0)),
            pl.BlockSpec((BLK, BITS), lambda j: (j, 0)),
        ],
        out_specs=[
            pl.BlockSpec((L, BITS), lambda j: (0, 0)),
            pl.BlockSpec((1, 1), lambda j: (0, 0)),
            pl.BlockSpec((1, 1), lambda j: (0, 0)),
        ],
        out_shape=[
            jax.ShapeDtypeStruct((L, BITS), jnp.float32),
            jax.ShapeDtypeStruct((1, 1), jnp.float32),
            jax.ShapeDtypeStruct((1, 1), jnp.float32),
        ],
        scratch_shapes=[pltpu.VMEM((L, 1), jnp.float32)],
    )(target_labels, trainlabels, traincodes, dB)
    return (aB, avg[0, 0], zero[0, 0])
